# Initial kernel scaffold; baseline (speedup 1.0000x reference)
import jax
import jax.numpy as jnp
from jax.experimental import pallas as pl


def kernel(x, conv1_w, conv1_shift, conv2_w, conv2_shift, conv3_w, conv3_shift, i3a_b1_w, i3a_b1_shift, i3a_b2a_w, i3a_b2a_shift, i3a_b2b_w, i3a_b2b_shift, i3a_b3a_w, i3a_b3a_shift, i3a_b3b_w, i3a_b3b_shift, i3a_b4_w, i3a_b4_shift, i3b_b1_w, i3b_b1_shift, i3b_b2a_w, i3b_b2a_shift, i3b_b2b_w, i3b_b2b_shift, i3b_b3a_w, i3b_b3a_shift, i3b_b3b_w, i3b_b3b_shift, i3b_b4_w, i3b_b4_shift, i4a_b1_w, i4a_b1_shift, i4a_b2a_w, i4a_b2a_shift, i4a_b2b_w, i4a_b2b_shift, i4a_b3a_w, i4a_b3a_shift, i4a_b3b_w, i4a_b3b_shift, i4a_b4_w, i4a_b4_shift, i4b_b1_w, i4b_b1_shift, i4b_b2a_w, i4b_b2a_shift, i4b_b2b_w, i4b_b2b_shift, i4b_b3a_w, i4b_b3a_shift, i4b_b3b_w, i4b_b3b_shift, i4b_b4_w, i4b_b4_shift, i4c_b1_w, i4c_b1_shift, i4c_b2a_w, i4c_b2a_shift, i4c_b2b_w, i4c_b2b_shift, i4c_b3a_w, i4c_b3a_shift, i4c_b3b_w, i4c_b3b_shift, i4c_b4_w, i4c_b4_shift, i4d_b1_w, i4d_b1_shift, i4d_b2a_w, i4d_b2a_shift, i4d_b2b_w, i4d_b2b_shift, i4d_b3a_w, i4d_b3a_shift, i4d_b3b_w, i4d_b3b_shift, i4d_b4_w, i4d_b4_shift, i4e_b1_w, i4e_b1_shift, i4e_b2a_w, i4e_b2a_shift, i4e_b2b_w, i4e_b2b_shift, i4e_b3a_w, i4e_b3a_shift, i4e_b3b_w, i4e_b3b_shift, i4e_b4_w, i4e_b4_shift, i5a_b1_w, i5a_b1_shift, i5a_b2a_w, i5a_b2a_shift, i5a_b2b_w, i5a_b2b_shift, i5a_b3a_w, i5a_b3a_shift, i5a_b3b_w, i5a_b3b_shift, i5a_b4_w, i5a_b4_shift, i5b_b1_w, i5b_b1_shift, i5b_b2a_w, i5b_b2a_shift, i5b_b2b_w, i5b_b2b_shift, i5b_b3a_w, i5b_b3a_shift, i5b_b3b_w, i5b_b3b_shift, i5b_b4_w, i5b_b4_shift, fc_w, fc_b):
    raise NotImplementedError("write your pallas kernel here")


# speedup vs baseline: 3.4816x; 3.4816x over previous
import functools
import numpy as np
import jax
import jax.numpy as jnp
from jax.experimental import pallas as pl
from jax.experimental.pallas import tpu as pltpu

_VMEM = 96 * 1024 * 1024


def _ceil_to(a, b):
    return -(-a // b) * b


def _mm_kernel(x_ref, w_ref, s_ref, o_ref):
    y = jnp.dot(x_ref[...], w_ref[...], preferred_element_type=jnp.float32)
    y = jnp.maximum(y + s_ref[...], 0.0)
    o_ref[...] = y.astype(o_ref.dtype)


def _mm1x1(x2d, w2d, shift):
    M, K = x2d.shape
    _, Ncol = w2d.shape
    Kp = _ceil_to(K, 128)
    Np = _ceil_to(Ncol, 128)
    tm = 512
    Mp = _ceil_to(M, tm)
    tn = 128 if Np > 512 else Np
    xb = x2d.astype(jnp.bfloat16)
    if (Mp, Kp) != (M, K):
        xb = jnp.pad(xb, ((0, Mp - M), (0, Kp - K)))
    wb = w2d.astype(jnp.bfloat16)
    if (Kp, Np) != (K, Ncol):
        wb = jnp.pad(wb, ((0, Kp - K), (0, Np - Ncol)))
    sb = shift.reshape(1, -1).astype(jnp.float32)
    if Np != Ncol:
        sb = jnp.pad(sb, ((0, 0), (0, Np - Ncol)))
    out = pl.pallas_call(
        _mm_kernel,
        grid=(Mp // tm, Np // tn),
        in_specs=[
            pl.BlockSpec((tm, Kp), lambda i, j: (i, 0)),
            pl.BlockSpec((Kp, tn), lambda i, j: (0, j)),
            pl.BlockSpec((1, tn), lambda i, j: (0, j)),
        ],
        out_specs=pl.BlockSpec((tm, tn), lambda i, j: (i, j)),
        out_shape=jax.ShapeDtypeStruct((Mp, Np), jnp.bfloat16),
        compiler_params=pltpu.CompilerParams(
            dimension_semantics=("parallel", "parallel"),
            vmem_limit_bytes=_VMEM),
    )(xb, wb, sb)
    if (Mp, Np) != (M, Ncol):
        out = out[:M, :Ncol]
    return out


def _c3_kernel(x_ref, w_ref, s_ref, o_ref, *, oh, ow):
    v = x_ref[...]
    wv = w_ref[...]
    bn = v.shape[0]
    cin = v.shape[3]
    acc = None
    for i in range(3):
        for j in range(3):
            p = v[:, i:i + oh, j:j + ow, :].reshape(bn * oh * ow, cin)
            d = jnp.dot(p, wv[i, j], preferred_element_type=jnp.float32)
            acc = d if acc is None else acc + d
    y = jnp.maximum(acc + s_ref[...], 0.0)
    o_ref[...] = y.reshape(bn, oh, ow, y.shape[-1]).astype(o_ref.dtype)


def _conv3x3(x, w, shift):
    N, H, W, Cin = x.shape
    Cout = w.shape[3]
    Np = _ceil_to(Cout, 128)
    cap = max(1, 1024 // (H * W))
    bn = 1
    while bn * 2 <= cap and N % (bn * 2) == 0:
        bn *= 2
    xp = jnp.pad(x.astype(jnp.bfloat16), ((0, 0), (1, 1), (1, 1), (0, 0)))
    wb = w.astype(jnp.bfloat16)
    sb = shift.reshape(1, -1).astype(jnp.float32)
    if Np != Cout:
        wb = jnp.pad(wb, ((0, 0), (0, 0), (0, 0), (0, Np - Cout)))
        sb = jnp.pad(sb, ((0, 0), (0, Np - Cout)))
    out = pl.pallas_call(
        functools.partial(_c3_kernel, oh=H, ow=W),
        grid=(N // bn,),
        in_specs=[
            pl.BlockSpec((bn, H + 2, W + 2, Cin), lambda n: (n, 0, 0, 0)),
            pl.BlockSpec((3, 3, Cin, Np), lambda n: (0, 0, 0, 0)),
            pl.BlockSpec((1, Np), lambda n: (0, 0)),
        ],
        out_specs=pl.BlockSpec((bn, H, W, Np), lambda n: (n, 0, 0, 0)),
        out_shape=jax.ShapeDtypeStruct((N, H, W, Np), jnp.bfloat16),
        compiler_params=pltpu.CompilerParams(
            dimension_semantics=("parallel",),
            vmem_limit_bytes=_VMEM),
    )(xp, wb, sb)
    if Np != Cout:
        out = out[..., :Cout]
    return out


def _pool_kernel(x_ref, o_ref, *, k, s, oh, ow):
    v = x_ref[...]
    bn, _, wp, c = v.shape
    rows = []
    for d in range(k):
        if s == 1:
            rows.append(v[:, d:d + oh])
        else:
            rows.append(v[:, d:d + 2 * oh].reshape(bn, oh, 2, wp, c)[:, :, 0])
    rm = rows[0]
    for r in rows[1:]:
        rm = jnp.maximum(rm, r)
    cols = []
    for d in range(k):
        if s == 1:
            cols.append(rm[:, :, d:d + ow])
        else:
            cols.append(rm[:, :, d:d + 2 * ow].reshape(bn, oh, ow, 2, c)[:, :, :, 0])
    cm = cols[0]
    for col in cols[1:]:
        cm = jnp.maximum(cm, col)
    o_ref[...] = cm


def _maxpool(x, k, s, padding):
    N, H, W, C = x.shape

    def out_dim(d):
        o = -(-(d + 2 * padding - k) // s) + 1
        if (o - 1) * s >= d + padding:
            o -= 1
        return o

    OH, OW = out_dim(H), out_dim(W)
    Hp = (k - 1) + s * OH
    Wp = (k - 1) + s * OW
    xb = x.astype(jnp.bfloat16)
    xp = jnp.pad(xb, ((0, 0), (padding, Hp - padding - H),
                      (padding, Wp - padding - W), (0, 0)),
                 constant_values=-np.inf)
    cap = max(1, 2048 // (OH * OW))
    bn = 1
    while bn * 2 <= cap and N % (bn * 2) == 0:
        bn *= 2
    out = pl.pallas_call(
        functools.partial(_pool_kernel, k=k, s=s, oh=OH, ow=OW),
        grid=(N // bn,),
        in_specs=[pl.BlockSpec((bn, Hp, Wp, C), lambda n: (n, 0, 0, 0))],
        out_specs=pl.BlockSpec((bn, OH, OW, C), lambda n: (n, 0, 0, 0)),
        out_shape=jax.ShapeDtypeStruct((N, OH, OW, C), jnp.bfloat16),
        compiler_params=pltpu.CompilerParams(
            dimension_semantics=("parallel",),
            vmem_limit_bytes=_VMEM),
    )(xp)
    return out


def _head_kernel(x_ref, w_ref, b_ref, o_ref, *, inv_hw):
    m = jnp.sum(x_ref[...].astype(jnp.float32), axis=1) * inv_hw
    y = jnp.dot(m.astype(jnp.bfloat16), w_ref[...],
                preferred_element_type=jnp.float32) + b_ref[...]
    o_ref[...] = y


def _head(x, w, b):
    N, H, W, C = x.shape
    HW = H * W
    ncls = w.shape[1]
    Np = _ceil_to(ncls, 128)
    xr = x.reshape(N, HW, C).astype(jnp.bfloat16)
    wb = jnp.pad(w.astype(jnp.bfloat16), ((0, 0), (0, Np - ncls)))
    bb = jnp.pad(b.reshape(1, -1).astype(jnp.float32), ((0, 0), (0, Np - ncls)))
    out = pl.pallas_call(
        functools.partial(_head_kernel, inv_hw=1.0 / HW),
        grid=(1,),
        in_specs=[
            pl.BlockSpec((N, HW, C), lambda i: (0, 0, 0)),
            pl.BlockSpec((C, Np), lambda i: (0, 0)),
            pl.BlockSpec((1, Np), lambda i: (0, 0)),
        ],
        out_specs=pl.BlockSpec((N, Np), lambda i: (0, 0)),
        out_shape=jax.ShapeDtypeStruct((N, Np), jnp.float32),
        compiler_params=pltpu.CompilerParams(
            dimension_semantics=("arbitrary",),
            vmem_limit_bytes=_VMEM),
    )(xr, wb, bb)
    return out[:, :ncls]


def _inception(x, b1w, b1s, b2aw, b2as, b2bw, b2bs, b3aw, b3as, b3bw, b3bs,
               b4w, b4s):
    N, H, W, C = x.shape
    xf = x.reshape(N * H * W, C)
    wcat = jnp.concatenate(
        [b1w.reshape(C, -1), b2aw.reshape(C, -1), b3aw.reshape(C, -1)], axis=1)
    scat = jnp.concatenate([b1s, b2as, b3as])
    y = _mm1x1(xf, wcat, scat)
    c1 = b1w.shape[-1]
    c2 = b2aw.shape[-1]
    b1 = y[:, :c1].reshape(N, H, W, c1)
    b2a = y[:, c1:c1 + c2].reshape(N, H, W, c2)
    b3a = y[:, c1 + c2:].reshape(N, H, W, -1)
    b2 = _conv3x3(b2a, b2bw, b2bs)
    b3 = _conv3x3(b3a, b3bw, b3bs)
    p = _maxpool(x, 3, 1, 1)
    b4 = _mm1x1(p.reshape(N * H * W, C), b4w.reshape(C, -1), b4s)
    b4 = b4.reshape(N, H, W, -1)
    return jnp.concatenate([b1, b2, b3, b4], axis=-1)


def kernel(x, conv1_w, conv1_shift, conv2_w, conv2_shift, conv3_w, conv3_shift, i3a_b1_w, i3a_b1_shift, i3a_b2a_w, i3a_b2a_shift, i3a_b2b_w, i3a_b2b_shift, i3a_b3a_w, i3a_b3a_shift, i3a_b3b_w, i3a_b3b_shift, i3a_b4_w, i3a_b4_shift, i3b_b1_w, i3b_b1_shift, i3b_b2a_w, i3b_b2a_shift, i3b_b2b_w, i3b_b2b_shift, i3b_b3a_w, i3b_b3a_shift, i3b_b3b_w, i3b_b3b_shift, i3b_b4_w, i3b_b4_shift, i4a_b1_w, i4a_b1_shift, i4a_b2a_w, i4a_b2a_shift, i4a_b2b_w, i4a_b2b_shift, i4a_b3a_w, i4a_b3a_shift, i4a_b3b_w, i4a_b3b_shift, i4a_b4_w, i4a_b4_shift, i4b_b1_w, i4b_b1_shift, i4b_b2a_w, i4b_b2a_shift, i4b_b2b_w, i4b_b2b_shift, i4b_b3a_w, i4b_b3a_shift, i4b_b3b_w, i4b_b3b_shift, i4b_b4_w, i4b_b4_shift, i4c_b1_w, i4c_b1_shift, i4c_b2a_w, i4c_b2a_shift, i4c_b2b_w, i4c_b2b_shift, i4c_b3a_w, i4c_b3a_shift, i4c_b3b_w, i4c_b3b_shift, i4c_b4_w, i4c_b4_shift, i4d_b1_w, i4d_b1_shift, i4d_b2a_w, i4d_b2a_shift, i4d_b2b_w, i4d_b2b_shift, i4d_b3a_w, i4d_b3a_shift, i4d_b3b_w, i4d_b3b_shift, i4d_b4_w, i4d_b4_shift, i4e_b1_w, i4e_b1_shift, i4e_b2a_w, i4e_b2a_shift, i4e_b2b_w, i4e_b2b_shift, i4e_b3a_w, i4e_b3a_shift, i4e_b3b_w, i4e_b3b_shift, i4e_b4_w, i4e_b4_shift, i5a_b1_w, i5a_b1_shift, i5a_b2a_w, i5a_b2a_shift, i5a_b2b_w, i5a_b2b_shift, i5a_b3a_w, i5a_b3a_shift, i5a_b3b_w, i5a_b3b_shift, i5a_b4_w, i5a_b4_shift, i5b_b1_w, i5b_b1_shift, i5b_b2a_w, i5b_b2a_shift, i5b_b2b_w, i5b_b2b_shift, i5b_b3a_w, i5b_b3a_shift, i5b_b3b_w, i5b_b3b_shift, i5b_b4_w, i5b_b4_shift, fc_w, fc_b):
    sc = jnp.array([0.229, 0.224, 0.225], jnp.float32) / 0.5
    sh = (jnp.array([0.485, 0.456, 0.406], jnp.float32) - 0.5) / 0.5
    w1 = conv1_w * sc.reshape(1, 1, 3, 1)
    s1 = conv1_shift + jnp.sum(conv1_w * sh.reshape(1, 1, 3, 1), axis=(0, 1, 2))

    xn = jnp.transpose(x, (0, 2, 3, 1)).astype(jnp.bfloat16)
    N = xn.shape[0]

    xp = jnp.pad(xn, ((0, 0), (3, 3), (3, 3), (0, 0)))
    cols = [xp[:, i:i + 223:2, j:j + 223:2, :]
            for i in range(7) for j in range(7)]
    patches = jnp.concatenate(cols, axis=-1).reshape(N * 112 * 112, 147)
    y = _mm1x1(patches, w1.reshape(147, -1), s1).reshape(N, 112, 112, 64)

    y = _maxpool(y, 3, 2, 0)
    y = _mm1x1(y.reshape(N * 56 * 56, 64), conv2_w.reshape(64, -1),
               conv2_shift).reshape(N, 56, 56, 64)
    y = _conv3x3(y, conv3_w, conv3_shift)
    y = _maxpool(y, 3, 2, 0)

    y = _inception(y, i3a_b1_w, i3a_b1_shift, i3a_b2a_w, i3a_b2a_shift,
                   i3a_b2b_w, i3a_b2b_shift, i3a_b3a_w, i3a_b3a_shift,
                   i3a_b3b_w, i3a_b3b_shift, i3a_b4_w, i3a_b4_shift)
    y = _inception(y, i3b_b1_w, i3b_b1_shift, i3b_b2a_w, i3b_b2a_shift,
                   i3b_b2b_w, i3b_b2b_shift, i3b_b3a_w, i3b_b3a_shift,
                   i3b_b3b_w, i3b_b3b_shift, i3b_b4_w, i3b_b4_shift)
    y = _maxpool(y, 3, 2, 0)
    y = _inception(y, i4a_b1_w, i4a_b1_shift, i4a_b2a_w, i4a_b2a_shift,
                   i4a_b2b_w, i4a_b2b_shift, i4a_b3a_w, i4a_b3a_shift,
                   i4a_b3b_w, i4a_b3b_shift, i4a_b4_w, i4a_b4_shift)
    y = _inception(y, i4b_b1_w, i4b_b1_shift, i4b_b2a_w, i4b_b2a_shift,
                   i4b_b2b_w, i4b_b2b_shift, i4b_b3a_w, i4b_b3a_shift,
                   i4b_b3b_w, i4b_b3b_shift, i4b_b4_w, i4b_b4_shift)
    y = _inception(y, i4c_b1_w, i4c_b1_shift, i4c_b2a_w, i4c_b2a_shift,
                   i4c_b2b_w, i4c_b2b_shift, i4c_b3a_w, i4c_b3a_shift,
                   i4c_b3b_w, i4c_b3b_shift, i4c_b4_w, i4c_b4_shift)
    y = _inception(y, i4d_b1_w, i4d_b1_shift, i4d_b2a_w, i4d_b2a_shift,
                   i4d_b2b_w, i4d_b2b_shift, i4d_b3a_w, i4d_b3a_shift,
                   i4d_b3b_w, i4d_b3b_shift, i4d_b4_w, i4d_b4_shift)
    y = _inception(y, i4e_b1_w, i4e_b1_shift, i4e_b2a_w, i4e_b2a_shift,
                   i4e_b2b_w, i4e_b2b_shift, i4e_b3a_w, i4e_b3a_shift,
                   i4e_b3b_w, i4e_b3b_shift, i4e_b4_w, i4e_b4_shift)
    y = _maxpool(y, 2, 2, 0)
    y = _inception(y, i5a_b1_w, i5a_b1_shift, i5a_b2a_w, i5a_b2a_shift,
                   i5a_b2b_w, i5a_b2b_shift, i5a_b3a_w, i5a_b3a_shift,
                   i5a_b3b_w, i5a_b3b_shift, i5a_b4_w, i5a_b4_shift)
    y = _inception(y, i5b_b1_w, i5b_b1_shift, i5b_b2a_w, i5b_b2a_shift,
                   i5b_b2b_w, i5b_b2b_shift, i5b_b3a_w, i5b_b3a_shift,
                   i5b_b3b_w, i5b_b3b_shift, i5b_b4_w, i5b_b4_shift)
    return _head(y, fc_w, fc_b)
